# baseline (speedup 1.0000x reference)
import functools

import jax
import jax.numpy as jnp
from jax import lax
from jax.experimental import pallas as pl
from jax.experimental.pallas import tpu as pltpu
from jax.experimental.pallas import tpu_sc as plsc

N_NODES = 10000
N_EDGES = 320000
D = 128
NW = 32
EDGES_PER_TILE = N_EDGES // NW
CHUNK = 80
N_CHUNKS = EDGES_PER_TILE // CHUNK
L = 16


def _edge_kernel(z_hbm, src_hbm, dst_hbm, out_hbm, z_sh,
                 idx_s, idx_d, rows_s, rows_d, out_b,
                 sems_is, sems_id, sems_s, sems_d, sems_o):
    sid = lax.axis_index("s")
    wid = sid * 2 + lax.axis_index("c")
    tile_base = wid * EDGES_PER_TILE
    lanes = lax.iota(jnp.int32, L)

    rows_share = 624
    pltpu.sync_copy(z_hbm.at[pl.ds(sid * rows_share, rows_share), :],
                    z_sh.at[pl.ds(sid * rows_share, rows_share), :])

    @pl.when(sid == 0)
    def _():
        rem = 16 * rows_share
        pltpu.sync_copy(z_hbm.at[pl.ds(rem, N_NODES - rem), :],
                        z_sh.at[pl.ds(rem, N_NODES - rem), :])

    plsc.subcore_barrier()

    def fire_idx(cidx, b):
        off = tile_base + cidx * CHUNK
        pltpu.async_copy(src_hbm.at[pl.ds(off, CHUNK)], idx_s[b], sems_is[b])
        pltpu.async_copy(dst_hbm.at[pl.ds(off, CHUNK)], idx_d[b], sems_id[b])

    def wait_idx(b):
        pltpu.make_async_copy(src_hbm.at[pl.ds(0, CHUNK)], idx_s[b],
                              sems_is[b]).wait()
        pltpu.make_async_copy(dst_hbm.at[pl.ds(0, CHUNK)], idx_d[b],
                              sems_id[b]).wait()

    def fire_rows(b):
        pltpu.async_copy(z_sh.at[idx_s[b]], rows_s[b], sems_s[b])
        pltpu.async_copy(z_sh.at[idx_d[b]], rows_d[b], sems_d[b])

    def drain_rows(b):
        pltpu.make_async_copy(z_sh.at[idx_s[b]], rows_s[b], sems_s[b]).wait()
        pltpu.make_async_copy(z_sh.at[idx_d[b]], rows_d[b], sems_d[b]).wait()

    def fire_out(cidx, b):
        off = tile_base + cidx * CHUNK
        pltpu.async_copy(out_b[b], out_hbm.at[pl.ds(off, CHUNK)], sems_o[b])

    def wait_out(b):
        pltpu.make_async_copy(out_b[b], out_hbm.at[pl.ds(0, CHUNK)],
                              sems_o[b]).wait()

    def compute(b):
        rs, rd, ob = rows_s[b], rows_d[b], out_b[b]

        def group_body(g, _):
            erow = lanes + g * L
            acc = jnp.zeros((L,), jnp.float32)
            for k in range(D):
                col = jnp.full((L,), k, jnp.int32)
                a = plsc.load_gather(rs, [erow, col])
                bb = plsc.load_gather(rd, [erow, col])
                acc = acc + a * bb
            y = 1.0 / (1.0 + jnp.exp(-acc))
            ob[pl.ds(g * L, L)] = y
            return _

        lax.fori_loop(0, CHUNK // L, group_body, None)

    def step(c, b):
        drain_rows(b)

        @pl.when(c + 2 < N_CHUNKS)
        def _():
            fire_idx(c + 2, b)

        wait_idx(1 - b)
        fire_rows(1 - b)

        @pl.when(c >= 2)
        def _():
            wait_out(b)

        compute(b)
        fire_out(c, b)

    fire_idx(0, 0)
    fire_idx(1, 1)
    wait_idx(0)
    fire_rows(0)

    def outer_body(c2, _):
        step(c2 * 2, 0)
        step(c2 * 2 + 1, 1)
        return _

    lax.fori_loop(0, (N_CHUNKS - 1) // 2, outer_body, None)

    last = N_CHUNKS - 1
    b = last % 2
    drain_rows(b)
    wait_out(b)
    compute(b)
    fire_out(last, b)
    wait_out(1 - b)
    wait_out(b)


@jax.jit
def _decode(z, src, dst):
    mesh = plsc.VectorSubcoreMesh(core_axis_name="c", subcore_axis_name="s")
    fn = functools.partial(
        pl.kernel,
        mesh=mesh,
        out_type=jax.ShapeDtypeStruct((N_EDGES,), jnp.float32),
        compiler_params=pltpu.CompilerParams(needs_layout_passes=False),
        scratch_types=[
            pltpu.VMEM_SHARED((N_NODES, D), jnp.float32),
            [pltpu.VMEM((CHUNK,), jnp.int32) for _ in range(2)],
            [pltpu.VMEM((CHUNK,), jnp.int32) for _ in range(2)],
            [pltpu.VMEM((CHUNK, D), jnp.float32) for _ in range(2)],
            [pltpu.VMEM((CHUNK, D), jnp.float32) for _ in range(2)],
            [pltpu.VMEM((CHUNK,), jnp.float32) for _ in range(2)],
            [pltpu.SemaphoreType.DMA for _ in range(2)],
            [pltpu.SemaphoreType.DMA for _ in range(2)],
            [pltpu.SemaphoreType.DMA for _ in range(2)],
            [pltpu.SemaphoreType.DMA for _ in range(2)],
            [pltpu.SemaphoreType.DMA for _ in range(2)],
        ],
    )(_edge_kernel)
    return fn(z, src, dst)


def kernel(z, edge_index):
    return _decode(z, edge_index[0], edge_index[1])


# speedup vs baseline: 1.0016x; 1.0016x over previous
import functools

import jax
import jax.numpy as jnp
from jax import lax
from jax.experimental import pallas as pl
from jax.experimental.pallas import tpu as pltpu
from jax.experimental.pallas import tpu_sc as plsc

N_NODES = 10000
N_EDGES = 320000
D = 128
NW = 32
EDGES_PER_TILE = N_EDGES // NW
CHUNK = 80
N_CHUNKS = EDGES_PER_TILE // CHUNK
L = 16


def _edge_kernel(z_hbm, src_hbm, dst_hbm, out_hbm, z_sh,
                 idx_s, idx_d, rows_s, rows_d, out_b,
                 sems_is, sems_id, sems_s, sems_d, sems_o):
    sid = lax.axis_index("s")
    wid = sid * 2 + lax.axis_index("c")
    tile_base = wid * EDGES_PER_TILE
    lanes = lax.iota(jnp.int32, L)

    rows_share = 624
    pltpu.sync_copy(z_hbm.at[pl.ds(sid * rows_share, rows_share), :],
                    z_sh.at[pl.ds(sid * rows_share, rows_share), :])

    @pl.when(sid == 0)
    def _():
        rem = 16 * rows_share
        pltpu.sync_copy(z_hbm.at[pl.ds(rem, N_NODES - rem), :],
                        z_sh.at[pl.ds(rem, N_NODES - rem), :])

    plsc.subcore_barrier()

    def fire_idx(cidx, b):
        off = tile_base + cidx * CHUNK
        pltpu.async_copy(src_hbm.at[pl.ds(off, CHUNK)], idx_s[b], sems_is[b])
        pltpu.async_copy(dst_hbm.at[pl.ds(off, CHUNK)], idx_d[b], sems_id[b])

    def wait_idx(b):
        pltpu.make_async_copy(src_hbm.at[pl.ds(0, CHUNK)], idx_s[b],
                              sems_is[b]).wait()
        pltpu.make_async_copy(dst_hbm.at[pl.ds(0, CHUNK)], idx_d[b],
                              sems_id[b]).wait()

    def fire_rows(b):
        pltpu.async_copy(z_sh.at[idx_s[b]], rows_s[b], sems_s[b])
        pltpu.async_copy(z_sh.at[idx_d[b]], rows_d[b], sems_d[b])

    def drain_rows(b):
        pltpu.make_async_copy(z_sh.at[idx_s[b]], rows_s[b], sems_s[b]).wait()
        pltpu.make_async_copy(z_sh.at[idx_d[b]], rows_d[b], sems_d[b]).wait()

    def fire_out(cidx, b):
        off = tile_base + cidx * CHUNK
        pltpu.async_copy(out_b[b], out_hbm.at[pl.ds(off, CHUNK)], sems_o[b])

    def wait_out(b):
        pltpu.make_async_copy(out_b[b], out_hbm.at[pl.ds(0, CHUNK)],
                              sems_o[b]).wait()

    def compute(b):
        rs, rd, ob = rows_s[b], rows_d[b], out_b[b]

        def group_body(g, _):
            erow = lanes + g * L

            def col_body(k, acc):
                col = jnp.full((L,), 1, jnp.int32) * k
                a = plsc.load_gather(rs, [erow, col])
                bb = plsc.load_gather(rd, [erow, col])
                return acc + a * bb

            acc = lax.fori_loop(0, D, col_body, jnp.zeros((L,), jnp.float32),
                                unroll=8)
            y = 1.0 / (1.0 + jnp.exp(-acc))
            ob[pl.ds(g * L, L)] = y
            return _

        lax.fori_loop(0, CHUNK // L, group_body, None)

    def step(c, b):
        drain_rows(b)

        @pl.when(c + 2 < N_CHUNKS)
        def _():
            fire_idx(c + 2, b)

        wait_idx(1 - b)
        fire_rows(1 - b)

        @pl.when(c >= 2)
        def _():
            wait_out(b)

        compute(b)
        fire_out(c, b)

    fire_idx(0, 0)
    fire_idx(1, 1)
    wait_idx(0)
    fire_rows(0)

    def outer_body(c2, _):
        step(c2 * 2, 0)
        step(c2 * 2 + 1, 1)
        return _

    lax.fori_loop(0, (N_CHUNKS - 1) // 2, outer_body, None)

    last = N_CHUNKS - 1
    b = last % 2
    drain_rows(b)
    wait_out(b)
    compute(b)
    fire_out(last, b)
    wait_out(1 - b)
    wait_out(b)


@jax.jit
def _decode(z, src, dst):
    mesh = plsc.VectorSubcoreMesh(core_axis_name="c", subcore_axis_name="s")
    fn = functools.partial(
        pl.kernel,
        mesh=mesh,
        out_type=jax.ShapeDtypeStruct((N_EDGES,), jnp.float32),
        compiler_params=pltpu.CompilerParams(needs_layout_passes=False),
        scratch_types=[
            pltpu.VMEM_SHARED((N_NODES, D), jnp.float32),
            [pltpu.VMEM((CHUNK,), jnp.int32) for _ in range(2)],
            [pltpu.VMEM((CHUNK,), jnp.int32) for _ in range(2)],
            [pltpu.VMEM((CHUNK, D), jnp.float32) for _ in range(2)],
            [pltpu.VMEM((CHUNK, D), jnp.float32) for _ in range(2)],
            [pltpu.VMEM((CHUNK,), jnp.float32) for _ in range(2)],
            [pltpu.SemaphoreType.DMA for _ in range(2)],
            [pltpu.SemaphoreType.DMA for _ in range(2)],
            [pltpu.SemaphoreType.DMA for _ in range(2)],
            [pltpu.SemaphoreType.DMA for _ in range(2)],
            [pltpu.SemaphoreType.DMA for _ in range(2)],
        ],
    )(_edge_kernel)
    return fn(z, src, dst)


def kernel(z, edge_index):
    return _decode(z, edge_index[0], edge_index[1])


# speedup vs baseline: 1.0025x; 1.0009x over previous
import functools

import jax
import jax.numpy as jnp
from jax import lax
from jax.experimental import pallas as pl
from jax.experimental.pallas import tpu as pltpu
from jax.experimental.pallas import tpu_sc as plsc

N_NODES = 10000
N_EDGES = 320000
D = 128
NW = 32
EDGES_PER_TILE = N_EDGES // NW
CHUNK = 80
N_CHUNKS = EDGES_PER_TILE // CHUNK
L = 16


def _edge_kernel(z_hbm, src_hbm, dst_hbm, out_hbm, z_sh,
                 idx_s, idx_d, rows_s, rows_d, out_b,
                 sems_is, sems_id, sems_s, sems_d, sems_o):
    sid = lax.axis_index("s")
    wid = sid * 2 + lax.axis_index("c")
    tile_base = wid * EDGES_PER_TILE
    lanes = lax.iota(jnp.int32, L)

    rows_share = 624
    pltpu.sync_copy(z_hbm.at[pl.ds(sid * rows_share, rows_share), :],
                    z_sh.at[pl.ds(sid * rows_share, rows_share), :])

    @pl.when(sid == 0)
    def _():
        rem = 16 * rows_share
        pltpu.sync_copy(z_hbm.at[pl.ds(rem, N_NODES - rem), :],
                        z_sh.at[pl.ds(rem, N_NODES - rem), :])

    plsc.subcore_barrier()

    def fire_idx(cidx, b):
        off = tile_base + cidx * CHUNK
        pltpu.async_copy(src_hbm.at[pl.ds(off, CHUNK)], idx_s[b], sems_is[b])
        pltpu.async_copy(dst_hbm.at[pl.ds(off, CHUNK)], idx_d[b], sems_id[b])

    def wait_idx(b):
        pltpu.make_async_copy(src_hbm.at[pl.ds(0, CHUNK)], idx_s[b],
                              sems_is[b]).wait()
        pltpu.make_async_copy(dst_hbm.at[pl.ds(0, CHUNK)], idx_d[b],
                              sems_id[b]).wait()

    H = CHUNK // 2

    def fire_rows(b):
        pltpu.async_copy(z_sh.at[idx_s[b].at[pl.ds(0, H)]],
                         rows_s[b].at[pl.ds(0, H), :], sems_s[b])
        pltpu.async_copy(z_sh.at[idx_s[b].at[pl.ds(H, H)]],
                         rows_s[b].at[pl.ds(H, H), :], sems_s[b])
        pltpu.async_copy(z_sh.at[idx_d[b].at[pl.ds(0, H)]],
                         rows_d[b].at[pl.ds(0, H), :], sems_d[b])
        pltpu.async_copy(z_sh.at[idx_d[b].at[pl.ds(H, H)]],
                         rows_d[b].at[pl.ds(H, H), :], sems_d[b])

    def drain_rows(b):
        for off in (0, H):
            pltpu.make_async_copy(z_sh.at[idx_s[b].at[pl.ds(off, H)]],
                                  rows_s[b].at[pl.ds(off, H), :],
                                  sems_s[b]).wait()
            pltpu.make_async_copy(z_sh.at[idx_d[b].at[pl.ds(off, H)]],
                                  rows_d[b].at[pl.ds(off, H), :],
                                  sems_d[b]).wait()

    def fire_out(cidx, b):
        off = tile_base + cidx * CHUNK
        pltpu.async_copy(out_b[b], out_hbm.at[pl.ds(off, CHUNK)], sems_o[b])

    def wait_out(b):
        pltpu.make_async_copy(out_b[b], out_hbm.at[pl.ds(0, CHUNK)],
                              sems_o[b]).wait()

    def compute(b):
        rs, rd, ob = rows_s[b], rows_d[b], out_b[b]

        def group_body(g, _):
            erow = lanes + g * L

            def col_body(k, acc):
                col = jnp.full((L,), 1, jnp.int32) * k
                a = plsc.load_gather(rs, [erow, col])
                bb = plsc.load_gather(rd, [erow, col])
                return acc + a * bb

            acc = lax.fori_loop(0, D, col_body, jnp.zeros((L,), jnp.float32),
                                unroll=8)
            y = 1.0 / (1.0 + jnp.exp(-acc))
            ob[pl.ds(g * L, L)] = y
            return _

        lax.fori_loop(0, CHUNK // L, group_body, None)

    def step(c, b):
        drain_rows(b)

        @pl.when(c + 2 < N_CHUNKS)
        def _():
            fire_idx(c + 2, b)

        wait_idx(1 - b)
        fire_rows(1 - b)

        @pl.when(c >= 2)
        def _():
            wait_out(b)

        compute(b)
        fire_out(c, b)

    fire_idx(0, 0)
    fire_idx(1, 1)
    wait_idx(0)
    fire_rows(0)

    def outer_body(c2, _):
        step(c2 * 2, 0)
        step(c2 * 2 + 1, 1)
        return _

    lax.fori_loop(0, (N_CHUNKS - 1) // 2, outer_body, None)

    last = N_CHUNKS - 1
    b = last % 2
    drain_rows(b)
    wait_out(b)
    compute(b)
    fire_out(last, b)
    wait_out(1 - b)
    wait_out(b)


@jax.jit
def _decode(z, src, dst):
    mesh = plsc.VectorSubcoreMesh(core_axis_name="c", subcore_axis_name="s")
    fn = functools.partial(
        pl.kernel,
        mesh=mesh,
        out_type=jax.ShapeDtypeStruct((N_EDGES,), jnp.float32),
        compiler_params=pltpu.CompilerParams(needs_layout_passes=False),
        scratch_types=[
            pltpu.VMEM_SHARED((N_NODES, D), jnp.float32),
            [pltpu.VMEM((CHUNK,), jnp.int32) for _ in range(2)],
            [pltpu.VMEM((CHUNK,), jnp.int32) for _ in range(2)],
            [pltpu.VMEM((CHUNK, D), jnp.float32) for _ in range(2)],
            [pltpu.VMEM((CHUNK, D), jnp.float32) for _ in range(2)],
            [pltpu.VMEM((CHUNK,), jnp.float32) for _ in range(2)],
            [pltpu.SemaphoreType.DMA for _ in range(2)],
            [pltpu.SemaphoreType.DMA for _ in range(2)],
            [pltpu.SemaphoreType.DMA for _ in range(2)],
            [pltpu.SemaphoreType.DMA for _ in range(2)],
            [pltpu.SemaphoreType.DMA for _ in range(2)],
        ],
    )(_edge_kernel)
    return fn(z, src, dst)


def kernel(z, edge_index):
    return _decode(z, edge_index[0], edge_index[1])


# speedup vs baseline: 1.7684x; 1.7640x over previous
import functools

import jax
import jax.numpy as jnp
from jax import lax
from jax.experimental import pallas as pl
from jax.experimental.pallas import tpu as pltpu
from jax.experimental.pallas import tpu_sc as plsc

N_NODES = 10000
N_EDGES = 320000
D = 128
D_PK = D // 2
NW = 32
EDGES_PER_TILE = N_EDGES // NW
CHUNK = 80
N_CHUNKS = EDGES_PER_TILE // CHUNK
NBUF = 4
L = 16


def _edge_kernel(z_hbm, src_hbm, dst_hbm, out_hbm, z_sh,
                 idx_s, idx_d, out_v, rows_s, rows_d, sems_s, sems_d):
    sid = lax.axis_index("s")
    wid = sid * 2 + lax.axis_index("c")
    tile_base = wid * EDGES_PER_TILE
    lanes = lax.iota(jnp.int32, L)

    rows_share = 624
    pltpu.sync_copy(z_hbm.at[pl.ds(sid * rows_share, rows_share), :],
                    z_sh.at[pl.ds(sid * rows_share, rows_share), :])

    @pl.when(sid == 0)
    def _():
        rem = 16 * rows_share
        pltpu.sync_copy(z_hbm.at[pl.ds(rem, N_NODES - rem), :],
                        z_sh.at[pl.ds(rem, N_NODES - rem), :])

    pltpu.sync_copy(src_hbm.at[pl.ds(tile_base, EDGES_PER_TILE)], idx_s)
    pltpu.sync_copy(dst_hbm.at[pl.ds(tile_base, EDGES_PER_TILE)], idx_d)
    plsc.subcore_barrier()

    def fire(cidx, b):
        off = cidx * CHUNK
        pltpu.async_copy(z_sh.at[idx_s.at[pl.ds(off, CHUNK)]], rows_s[b],
                         sems_s[b])
        pltpu.async_copy(z_sh.at[idx_d.at[pl.ds(off, CHUNK)]], rows_d[b],
                         sems_d[b])

    def drain(cidx, b):
        off = cidx * CHUNK
        pltpu.make_async_copy(z_sh.at[idx_s.at[pl.ds(off, CHUNK)]],
                              rows_s[b], sems_s[b]).wait()
        pltpu.make_async_copy(z_sh.at[idx_d.at[pl.ds(off, CHUNK)]],
                              rows_d[b], sems_d[b]).wait()

    def compute(cidx, b):
        rs, rd = rows_s[b], rows_d[b]

        def group_body(g, _):
            erow = lanes + g * L

            def col_body(k, carry):
                acc0, acc1 = carry
                col = jnp.full((L,), 1, jnp.int32) * k
                ai = plsc.load_gather(rs, [erow, col])
                bi = plsc.load_gather(rd, [erow, col])
                av = plsc.bitcast(ai, jnp.bfloat16)
                bv = plsc.bitcast(bi, jnp.bfloat16)
                p0, p1 = plsc.unpack(av * bv,
                                     format=plsc.PackFormat.INTERLEAVED,
                                     preferred_element_type=jnp.float32)
                return acc0 + p0, acc1 + p1

            acc0, acc1 = lax.fori_loop(
                0, D_PK, col_body,
                (jnp.zeros((L,), jnp.float32), jnp.zeros((L,), jnp.float32)),
                unroll=8)
            acc = acc0 + acc1
            y = 1.0 / (1.0 + jnp.exp(-acc))
            out_v[pl.ds(cidx * CHUNK + g * L, L)] = y
            return _

        lax.fori_loop(0, CHUNK // L, group_body, None)

    for c in range(NBUF - 1):
        fire(c, c)

    def outer_body(c4, _):
        for b in range(NBUF):
            cidx = c4 * NBUF + b
            nxt = cidx + (NBUF - 1)

            @pl.when(nxt < N_CHUNKS)
            def _():
                fire(nxt, (b + NBUF - 1) % NBUF)

            drain(cidx, b)
            compute(cidx, b)
        return _

    M = ((N_CHUNKS - 1) // NBUF) * NBUF
    lax.fori_loop(0, M // NBUF, outer_body, None)
    for cidx in range(M, N_CHUNKS):
        b = cidx % NBUF
        nxt = cidx + (NBUF - 1)
        if nxt < N_CHUNKS:
            fire(nxt, nxt % NBUF)
        drain(cidx, b)
        compute(cidx, b)

    pltpu.sync_copy(out_v, out_hbm.at[pl.ds(tile_base, EDGES_PER_TILE)])


@jax.jit
def _decode(z, src, dst):
    z_pk = jax.lax.bitcast_convert_type(
        z.astype(jnp.bfloat16).reshape(N_NODES, D_PK, 2), jnp.int32)
    mesh = plsc.VectorSubcoreMesh(core_axis_name="c", subcore_axis_name="s")
    fn = functools.partial(
        pl.kernel,
        mesh=mesh,
        out_type=jax.ShapeDtypeStruct((N_EDGES,), jnp.float32),
        compiler_params=pltpu.CompilerParams(needs_layout_passes=False,
                                             use_tc_tiling_on_sc=False),
        scratch_types=[
            pltpu.VMEM_SHARED((N_NODES, D_PK), jnp.int32),
            pltpu.VMEM((EDGES_PER_TILE,), jnp.int32),
            pltpu.VMEM((EDGES_PER_TILE,), jnp.int32),
            pltpu.VMEM((EDGES_PER_TILE,), jnp.float32),
            [pltpu.VMEM((CHUNK, D_PK), jnp.int32) for _ in range(NBUF)],
            [pltpu.VMEM((CHUNK, D_PK), jnp.int32) for _ in range(NBUF)],
            [pltpu.SemaphoreType.DMA for _ in range(NBUF)],
            [pltpu.SemaphoreType.DMA for _ in range(NBUF)],
        ],
    )(_edge_kernel)
    return fn(z_pk, src, dst)


def kernel(z, edge_index):
    return _decode(z, edge_index[0], edge_index[1])


# speedup vs baseline: 1.7701x; 1.0010x over previous
import functools

import jax
import jax.numpy as jnp
from jax import lax
from jax.experimental import pallas as pl
from jax.experimental.pallas import tpu as pltpu
from jax.experimental.pallas import tpu_sc as plsc

N_NODES = 10000
N_EDGES = 320000
D = 128
D_PK = D // 2
NW = 32
EDGES_PER_TILE = N_EDGES // NW
CHUNK = 80
N_CHUNKS = EDGES_PER_TILE // CHUNK
NBUF = 4
L = 16


def _edge_kernel(z_hbm, src_hbm, dst_hbm, out_hbm, z_sh,
                 idx_s, idx_d, out_v, rows_s, rows_d, sems_s, sems_d):
    sid = lax.axis_index("s")
    wid = sid * 2 + lax.axis_index("c")
    tile_base = wid * EDGES_PER_TILE
    lanes = lax.iota(jnp.int32, L)

    rows_share = 624
    pltpu.sync_copy(z_hbm.at[pl.ds(sid * rows_share, rows_share), :],
                    z_sh.at[pl.ds(sid * rows_share, rows_share), :])

    @pl.when(sid == 0)
    def _():
        rem = 16 * rows_share
        pltpu.sync_copy(z_hbm.at[pl.ds(rem, N_NODES - rem), :],
                        z_sh.at[pl.ds(rem, N_NODES - rem), :])

    pltpu.sync_copy(src_hbm.at[pl.ds(tile_base, EDGES_PER_TILE)], idx_s)
    pltpu.sync_copy(dst_hbm.at[pl.ds(tile_base, EDGES_PER_TILE)], idx_d)
    plsc.subcore_barrier()

    def fire(cidx, b):
        zt = z_sh if b % 2 == 0 else z_hbm
        off = cidx * CHUNK
        pltpu.async_copy(zt.at[idx_s.at[pl.ds(off, CHUNK)]], rows_s[b],
                         sems_s[b])
        pltpu.async_copy(zt.at[idx_d.at[pl.ds(off, CHUNK)]], rows_d[b],
                         sems_d[b])

    def drain(cidx, b):
        zt = z_sh if b % 2 == 0 else z_hbm
        off = cidx * CHUNK
        pltpu.make_async_copy(zt.at[idx_s.at[pl.ds(off, CHUNK)]],
                              rows_s[b], sems_s[b]).wait()
        pltpu.make_async_copy(zt.at[idx_d.at[pl.ds(off, CHUNK)]],
                              rows_d[b], sems_d[b]).wait()

    def compute(cidx, b):
        rs, rd = rows_s[b], rows_d[b]

        def group_body(g, _):
            erow = lanes + g * L

            def col_body(k, carry):
                acc0, acc1 = carry
                col = jnp.full((L,), 1, jnp.int32) * k
                ai = plsc.load_gather(rs, [erow, col])
                bi = plsc.load_gather(rd, [erow, col])
                av = plsc.bitcast(ai, jnp.bfloat16)
                bv = plsc.bitcast(bi, jnp.bfloat16)
                p0, p1 = plsc.unpack(av * bv,
                                     format=plsc.PackFormat.INTERLEAVED,
                                     preferred_element_type=jnp.float32)
                return acc0 + p0, acc1 + p1

            acc0, acc1 = lax.fori_loop(
                0, D_PK, col_body,
                (jnp.zeros((L,), jnp.float32), jnp.zeros((L,), jnp.float32)),
                unroll=8)
            acc = acc0 + acc1
            y = 1.0 / (1.0 + jnp.exp(-acc))
            out_v[pl.ds(cidx * CHUNK + g * L, L)] = y
            return _

        lax.fori_loop(0, CHUNK // L, group_body, None)

    for c in range(NBUF - 1):
        fire(c, c)

    def outer_body(c4, _):
        for b in range(NBUF):
            cidx = c4 * NBUF + b
            nxt = cidx + (NBUF - 1)

            @pl.when(nxt < N_CHUNKS)
            def _():
                fire(nxt, (b + NBUF - 1) % NBUF)

            drain(cidx, b)
            compute(cidx, b)
        return _

    M = ((N_CHUNKS - 1) // NBUF) * NBUF
    lax.fori_loop(0, M // NBUF, outer_body, None)
    for cidx in range(M, N_CHUNKS):
        b = cidx % NBUF
        nxt = cidx + (NBUF - 1)
        if nxt < N_CHUNKS:
            fire(nxt, nxt % NBUF)
        drain(cidx, b)
        compute(cidx, b)

    pltpu.sync_copy(out_v, out_hbm.at[pl.ds(tile_base, EDGES_PER_TILE)])


@jax.jit
def _decode(z, src, dst):
    z_pk = jax.lax.bitcast_convert_type(
        z.astype(jnp.bfloat16).reshape(N_NODES, D_PK, 2), jnp.int32)
    mesh = plsc.VectorSubcoreMesh(core_axis_name="c", subcore_axis_name="s")
    fn = functools.partial(
        pl.kernel,
        mesh=mesh,
        out_type=jax.ShapeDtypeStruct((N_EDGES,), jnp.float32),
        compiler_params=pltpu.CompilerParams(needs_layout_passes=False,
                                             use_tc_tiling_on_sc=False),
        scratch_types=[
            pltpu.VMEM_SHARED((N_NODES, D_PK), jnp.int32),
            pltpu.VMEM((EDGES_PER_TILE,), jnp.int32),
            pltpu.VMEM((EDGES_PER_TILE,), jnp.int32),
            pltpu.VMEM((EDGES_PER_TILE,), jnp.float32),
            [pltpu.VMEM((CHUNK, D_PK), jnp.int32) for _ in range(NBUF)],
            [pltpu.VMEM((CHUNK, D_PK), jnp.int32) for _ in range(NBUF)],
            [pltpu.SemaphoreType.DMA for _ in range(NBUF)],
            [pltpu.SemaphoreType.DMA for _ in range(NBUF)],
        ],
    )(_edge_kernel)
    return fn(z_pk, src, dst)


def kernel(z, edge_index):
    return _decode(z, edge_index[0], edge_index[1])


# speedup vs baseline: 1.7745x; 1.0025x over previous
import functools

import jax
import jax.numpy as jnp
from jax import lax
from jax.experimental import pallas as pl
from jax.experimental.pallas import tpu as pltpu
from jax.experimental.pallas import tpu_sc as plsc

N_NODES = 10000
N_EDGES = 320000
D = 128
D_PK = D // 2
NW = 32
EDGES_PER_TILE = N_EDGES // NW
CHUNK = 80
N_CHUNKS = EDGES_PER_TILE // CHUNK
NBUF = 6
L = 16


def _edge_kernel(z_hbm, src_hbm, dst_hbm, out_hbm,
                 idx_s, idx_d, out_v, rows_s, rows_d, sems_s, sems_d):
    sid = lax.axis_index("s")
    wid = sid * 2 + lax.axis_index("c")
    tile_base = wid * EDGES_PER_TILE
    lanes = lax.iota(jnp.int32, L)

    pltpu.sync_copy(src_hbm.at[pl.ds(tile_base, EDGES_PER_TILE)], idx_s)
    pltpu.sync_copy(dst_hbm.at[pl.ds(tile_base, EDGES_PER_TILE)], idx_d)

    def fire(cidx, b):
        off = cidx * CHUNK
        pltpu.async_copy(z_hbm.at[idx_s.at[pl.ds(off, CHUNK)]], rows_s[b],
                         sems_s[b])
        pltpu.async_copy(z_hbm.at[idx_d.at[pl.ds(off, CHUNK)]], rows_d[b],
                         sems_d[b])

    def drain(cidx, b):
        off = cidx * CHUNK
        pltpu.make_async_copy(z_hbm.at[idx_s.at[pl.ds(off, CHUNK)]],
                              rows_s[b], sems_s[b]).wait()
        pltpu.make_async_copy(z_hbm.at[idx_d.at[pl.ds(off, CHUNK)]],
                              rows_d[b], sems_d[b]).wait()

    def compute(cidx, b):
        rs, rd = rows_s[b], rows_d[b]

        def group_body(g, _):
            erow = lanes + g * L

            def col_body(k, carry):
                acc0, acc1 = carry
                col = jnp.full((L,), 1, jnp.int32) * k
                ai = plsc.load_gather(rs, [erow, col])
                bi = plsc.load_gather(rd, [erow, col])
                av = plsc.bitcast(ai, jnp.bfloat16)
                bv = plsc.bitcast(bi, jnp.bfloat16)
                p0, p1 = plsc.unpack(av * bv,
                                     format=plsc.PackFormat.INTERLEAVED,
                                     preferred_element_type=jnp.float32)
                return acc0 + p0, acc1 + p1

            acc0, acc1 = lax.fori_loop(
                0, D_PK, col_body,
                (jnp.zeros((L,), jnp.float32), jnp.zeros((L,), jnp.float32)),
                unroll=8)
            acc = acc0 + acc1
            y = 1.0 / (1.0 + jnp.exp(-acc))
            out_v[pl.ds(cidx * CHUNK + g * L, L)] = y
            return _

        lax.fori_loop(0, CHUNK // L, group_body, None)

    for c in range(NBUF - 1):
        fire(c, c)

    def outer_body(c4, _):
        for b in range(NBUF):
            cidx = c4 * NBUF + b
            nxt = cidx + (NBUF - 1)

            @pl.when(nxt < N_CHUNKS)
            def _():
                fire(nxt, (b + NBUF - 1) % NBUF)

            drain(cidx, b)
            compute(cidx, b)
        return _

    M = ((N_CHUNKS - 1) // NBUF) * NBUF
    lax.fori_loop(0, M // NBUF, outer_body, None)
    for cidx in range(M, N_CHUNKS):
        b = cidx % NBUF
        nxt = cidx + (NBUF - 1)
        if nxt < N_CHUNKS:
            fire(nxt, nxt % NBUF)
        drain(cidx, b)
        compute(cidx, b)

    pltpu.sync_copy(out_v, out_hbm.at[pl.ds(tile_base, EDGES_PER_TILE)])


@jax.jit
def _decode(z, src, dst):
    z_pk = jax.lax.bitcast_convert_type(
        z.astype(jnp.bfloat16).reshape(N_NODES, D_PK, 2), jnp.int32)
    mesh = plsc.VectorSubcoreMesh(core_axis_name="c", subcore_axis_name="s")
    fn = functools.partial(
        pl.kernel,
        mesh=mesh,
        out_type=jax.ShapeDtypeStruct((N_EDGES,), jnp.float32),
        compiler_params=pltpu.CompilerParams(needs_layout_passes=False,
                                             use_tc_tiling_on_sc=False),
        scratch_types=[
            pltpu.VMEM((EDGES_PER_TILE,), jnp.int32),
            pltpu.VMEM((EDGES_PER_TILE,), jnp.int32),
            pltpu.VMEM((EDGES_PER_TILE,), jnp.float32),
            [pltpu.VMEM((CHUNK, D_PK), jnp.int32) for _ in range(NBUF)],
            [pltpu.VMEM((CHUNK, D_PK), jnp.int32) for _ in range(NBUF)],
            [pltpu.SemaphoreType.DMA for _ in range(NBUF)],
            [pltpu.SemaphoreType.DMA for _ in range(NBUF)],
        ],
    )(_edge_kernel)
    return fn(z_pk, src, dst)


def kernel(z, edge_index):
    return _decode(z, edge_index[0], edge_index[1])
